# baseline (speedup 1.0000x reference)
import functools

import jax
import jax.numpy as jnp
from jax import lax
from jax.experimental import pallas as pl
from jax.experimental.pallas import tpu as pltpu
from jax.experimental.pallas import tpu_sc as plsc

N_NODES = 10000
N_EDGES = 320000
D = 128
NP = 10240
NROW = NP // 128
DUMP = 10200

NC, NS, L = 2, 16, 16
NW = NC * NS
WROWS = 80
EROWS = NW * WROWS
EPAD = EROWS * 128 - N_EDGES
SLICE = NP // NS


def _mesh():
    return plsc.VectorSubcoreMesh(core_axis_name="c", subcore_axis_name="s",
                                  num_cores=NC, num_subcores=NS)


def _qmap_body(q_ref, out_ref):
    node = (lax.broadcasted_iota(jnp.int32, (NROW, D), 0) * D
            + lax.broadcasted_iota(jnp.int32, (NROW, D), 1))
    acc = jnp.zeros((NROW, D), jnp.int32)
    for j in range(5):
        acc = acc | ((node == q_ref[j]).astype(jnp.int32) << j)
    out_ref[...] = acc


def _qmap(q8):
    return pl.pallas_call(
        _qmap_body,
        in_specs=[pl.BlockSpec(memory_space=pltpu.SMEM)],
        out_shape=jax.ShapeDtypeStruct((NROW, D), jnp.int32),
    )(q8)


def _passA_body(row2d, col2d, qmap_hbm, deg_out, mark_out,
                deg_acc, mark_acc, qv, rowb, colb, c128, idxb, onesb, zb,
                sem_d, sem_m):
    cid = lax.axis_index("c")
    sid = lax.axis_index("s")
    wid = cid * NS + sid

    ones16 = jnp.ones((L,), jnp.float32)
    zero16 = jnp.zeros((L,), jnp.float32)
    for k in range(8):
        onesb[pl.ds(k * L, L)] = ones16

    def zfill(i, c):
        zb[pl.ds(i * L, L)] = zero16
        return c
    lax.fori_loop(0, SLICE // L, zfill, 0)
    pltpu.sync_copy(zb, deg_acc.at[pl.ds(sid * SLICE, SLICE)])
    pltpu.sync_copy(zb, mark_acc.at[pl.ds(sid * SLICE, SLICE)])
    pltpu.sync_copy(qmap_hbm, qv)
    plsc.subcore_barrier()

    rb = wid * WROWS
    pltpu.sync_copy(row2d.at[pl.ds(rb, WROWS)], rowb)
    pltpu.sync_copy(col2d.at[pl.ds(rb, WROWS)], colb)

    def body(r, carry):
        cnt = jnp.int32(0)
        for k in range(8):
            cv = colb[r, pl.ds(k * L, L)]
            rv = rowb[r, pl.ds(k * L, L)]
            c128[pl.ds(k * L, L)] = cv
            hit = plsc.load_gather(qv, [cv]) > 0
            idxb[pl.ds(k * L, L)] = jnp.where(hit, rv, DUMP)
            cnt = cnt + jnp.sum(hit.astype(jnp.int32))
        cp = pltpu.async_copy(onesb, deg_acc.at[c128], sem_d, add=True)

        @pl.when(cnt > 0)
        def _():
            pltpu.async_copy(onesb, mark_acc.at[idxb], sem_m, add=True).wait()
        cp.wait()
        return carry
    lax.fori_loop(0, WROWS, body, 0)
    plsc.subcore_barrier()

    sl = pl.ds(sid * SLICE, SLICE)
    osl = pl.ds(cid * NP + sid * SLICE, SLICE)
    pltpu.sync_copy(deg_acc.at[sl], deg_out.at[osl])
    pltpu.sync_copy(mark_acc.at[sl], mark_out.at[osl])


def _passA(row2d, col2d, qmap):
    f = pl.kernel(
        _passA_body,
        out_type=[jax.ShapeDtypeStruct((NC * NP,), jnp.float32),
                  jax.ShapeDtypeStruct((NC * NP,), jnp.float32)],
        mesh=_mesh(),
        name="sc_passA",
        compiler_params=pltpu.CompilerParams(needs_layout_passes=False),
        scratch_types=[
            pltpu.VMEM_SHARED((NP,), jnp.float32),
            pltpu.VMEM_SHARED((NP,), jnp.float32),
            pltpu.VMEM((NP,), jnp.int32),
            pltpu.VMEM((WROWS, 128), jnp.int32),
            pltpu.VMEM((WROWS, 128), jnp.int32),
            pltpu.VMEM((128,), jnp.int32),
            pltpu.VMEM((128,), jnp.int32),
            pltpu.VMEM((128,), jnp.float32),
            pltpu.VMEM((SLICE,), jnp.float32),
            pltpu.SemaphoreType.DMA,
            pltpu.SemaphoreType.DMA,
        ],
    )
    return f(row2d, col2d, qmap)


LCAP_B = 4096
FLUSH_B = LCAP_B - 16 * 128 - 320 - 64
WIN = 16
NWIN = WROWS // WIN


def _drain(lrow, lsc, ltg, src_hbm, rows, tgtb, acc, cnt, sem_g, sem_s,
           dump_tg, br):
    zero16 = jnp.zeros((L,), jnp.float32)
    dump16 = jnp.full((L,), dump_tg, jnp.int32)
    izero16 = jnp.zeros((L,), jnp.int32)
    for k in range(br // L):
        lrow[pl.ds(cnt + k * L, L)] = izero16
        lsc[pl.ds(cnt + k * L, L)] = zero16
        ltg[pl.ds(cnt + k * L, L)] = dump16

    nb = (cnt + br - 1) // br

    def gstep(b, c):
        cp = pltpu.async_copy(src_hbm.at[lrow.at[pl.ds(b * br, br)]],
                              rows, sem_g)
        for k in range(br // L):
            tgtb[pl.ds(k * L, L)] = ltg[pl.ds(b * br + k * L, L)]
        cp.wait()

        def scale_row(i, cc):
            sb = plsc.load_gather(lsc, [jnp.full((L,), b * br + i, jnp.int32)])
            for k in range(8):
                rows[i, pl.ds(k * L, L)] = rows[i, pl.ds(k * L, L)] * sb
            return cc
        lax.fori_loop(0, br, scale_row, 0)
        pltpu.async_copy(rows, acc.at[tgtb], sem_s, add=True).wait()
        return c
    lax.fori_loop(0, nb, gstep, 0)


def _zero_rows(rows, nr):
    zero16 = jnp.zeros((L,), jnp.float32)

    def zf(i, c):
        for k in range(8):
            rows[i, pl.ds(k * L, L)] = zero16
        return c
    lax.fori_loop(0, nr, zf, 0)


def _passB_body(row2d, col2d, dismk_hbm, gx_hbm, agg_out,
                agg_acc, dmv, rowb, colb, lrow, lsc, ltg, rows, tgtb,
                sem_g, sem_s):
    cid = lax.axis_index("c")
    sid = lax.axis_index("s")
    wid = cid * NS + sid

    _zero_rows(rows, 64)
    pltpu.sync_copy(dismk_hbm, dmv)

    dump16 = jnp.full((L,), DUMP, jnp.int32)

    def zscan(t, cnt):
        off = sid * SLICE + t * L
        mk = dmv[pl.ds(off, L)] > 0
        n = jnp.sum(mk.astype(jnp.int32))

        @pl.when(n > 0)
        def _():
            vv = lax.iota(jnp.int32, L) + off
            plsc.store_compressed(lrow.at[pl.ds(cnt, L)], vv, mask=mk)
        return cnt + n
    cnt0 = lax.fori_loop(0, SLICE // L, zscan, jnp.int32(0))
    for k in range(4):
        lrow[pl.ds(cnt0 + k * L, L)] = dump16

    def zstep(b, c):
        for k in range(4):
            tgtb[pl.ds(k * L, L)] = lrow[pl.ds(b * 64 + k * L, L)]
        pltpu.async_copy(rows, agg_acc.at[tgtb], sem_s).wait()
        return c
    lax.fori_loop(0, (cnt0 + 63) // 64, zstep, 0)
    plsc.subcore_barrier()

    rb = wid * WROWS

    def win(w, cnt):
        pltpu.sync_copy(row2d.at[pl.ds(rb + w * WIN, WIN)], rowb)
        pltpu.sync_copy(col2d.at[pl.ds(rb + w * WIN, WIN)], colb)

        def cstep(r, cnt):
            for k in range(8):
                cv = colb[r, pl.ds(k * L, L)]
                rv = rowb[r, pl.ds(k * L, L)]
                dc = plsc.load_gather(dmv, [cv])
                mk = dc > 0
                n = jnp.sum(mk.astype(jnp.int32))

                @pl.when(n > 0)
                def _():
                    dr = jnp.abs(plsc.load_gather(dmv, [rv]))
                    plsc.store_compressed(lrow.at[pl.ds(cnt, L)], rv, mask=mk)
                    plsc.store_compressed(lsc.at[pl.ds(cnt, L)], dr * dc,
                                          mask=mk)
                    plsc.store_compressed(ltg.at[pl.ds(cnt, L)], cv, mask=mk)
                cnt = cnt + n
            return cnt
        cnt = lax.fori_loop(0, WIN, cstep, cnt)

        @pl.when(cnt > FLUSH_B)
        def _():
            _drain(lrow, lsc, ltg, gx_hbm, rows, tgtb, agg_acc, cnt,
                   sem_g, sem_s, DUMP, 64)
        return jnp.where(cnt > FLUSH_B, 0, cnt)
    cnt = lax.fori_loop(0, NWIN, win, jnp.int32(0))

    base = wid * (NP // NW)

    def sstep(t, cnt):
        off = base + t * L
        dv = dmv[pl.ds(off, L)]
        mk = dv > 0
        n = jnp.sum(mk.astype(jnp.int32))

        @pl.when(n > 0)
        def _():
            vv = lax.iota(jnp.int32, L) + off
            plsc.store_compressed(lrow.at[pl.ds(cnt, L)], vv, mask=mk)
            plsc.store_compressed(lsc.at[pl.ds(cnt, L)], dv * dv, mask=mk)
            plsc.store_compressed(ltg.at[pl.ds(cnt, L)], vv, mask=mk)
        return cnt + n
    cnt = lax.fori_loop(0, (NP // NW) // L, sstep, cnt)

    _drain(lrow, lsc, ltg, gx_hbm, rows, tgtb, agg_acc, cnt, sem_g, sem_s,
           DUMP, 64)
    plsc.subcore_barrier()
    for t in range(SLICE // 64):
        sl = pl.ds(sid * SLICE + t * 64, 64)
        pltpu.sync_copy(agg_acc.at[sl, :], agg_out.at[cid, sl, :])


def _passB(row2d, col2d, dismk, graph_x):
    f = pl.kernel(
        _passB_body,
        out_type=jax.ShapeDtypeStruct((NC, NP, D), jnp.float32),
        mesh=_mesh(),
        name="sc_passB",
        compiler_params=pltpu.CompilerParams(needs_layout_passes=False),
        scratch_types=[
            pltpu.VMEM_SHARED((NP, D), jnp.float32),
            pltpu.VMEM((NP,), jnp.float32),
            pltpu.VMEM((WIN, 128), jnp.int32),
            pltpu.VMEM((WIN, 128), jnp.int32),
            pltpu.VMEM((LCAP_B,), jnp.int32),
            pltpu.VMEM((LCAP_B,), jnp.float32),
            pltpu.VMEM((LCAP_B,), jnp.int32),
            pltpu.VMEM((64, D), jnp.float32),
            pltpu.VMEM((64,), jnp.int32),
            pltpu.SemaphoreType.DMA,
            pltpu.SemaphoreType.DMA,
        ],
    )
    return f(row2d, col2d, dismk, graph_x)


LCAP_C = 8192
FLUSH_AT = LCAP_C - 16 * 128 - 128


def _drain_c(lrow, lsc, lqm, src_hbm, rows, tgtb, acc, cnt, sem_g, sem_s):
    zero16 = jnp.zeros((L,), jnp.float32)
    izero16 = jnp.zeros((L,), jnp.int32)
    for k in range(8):
        lrow[pl.ds(cnt + k * L, L)] = izero16
        lsc[pl.ds(cnt + k * L, L)] = zero16
        lqm[pl.ds(cnt + k * L, L)] = izero16

    nb = (cnt + 127) // 128

    def gstep(b, c):
        cp = pltpu.async_copy(src_hbm.at[lrow.at[pl.ds(b * 128, 128)]],
                              rows, sem_g)
        cp.wait()

        def scale_row(i, cc):
            sb = plsc.load_gather(lsc, [jnp.full((L,), b * 128 + i, jnp.int32)])
            for k in range(8):
                rows[i, pl.ds(k * L, L)] = rows[i, pl.ds(k * L, L)] * sb
            return cc
        lax.fori_loop(0, 128, scale_row, 0)
        for j in range(5):
            for k in range(8):
                qmb = lqm[pl.ds(b * 128 + k * L, L)]
                tgtb[pl.ds(k * L, L)] = jnp.where(((qmb >> j) & 1) > 0, j, 7)
            pltpu.async_copy(rows, acc.at[tgtb], sem_s, add=True).wait()
        return c
    lax.fori_loop(0, nb, gstep, 0)


def _passC_body(row2d, col2d, dis_hbm, qmap_hbm, h1_hbm, agg2_out,
                agg2_acc, disv, qv, rowb, colb, lrow, lsc, ltg, rows, tgt128,
                sem_g, sem_s):
    cid = lax.axis_index("c")
    sid = lax.axis_index("s")
    wid = cid * NS + sid

    _zero_rows(rows, 128)

    @pl.when(sid == 0)
    def _():
        pltpu.sync_copy(rows.at[pl.ds(0, 8), :], agg2_acc)
    pltpu.sync_copy(dis_hbm, disv)
    pltpu.sync_copy(qmap_hbm, qv)
    plsc.subcore_barrier()

    rb = wid * WROWS

    def win(w, cnt):
        pltpu.sync_copy(row2d.at[pl.ds(rb + w * WIN, WIN)], rowb)
        pltpu.sync_copy(col2d.at[pl.ds(rb + w * WIN, WIN)], colb)

        def cstep(r, cnt):
            for k in range(8):
                cv = colb[r, pl.ds(k * L, L)]
                rv = rowb[r, pl.ds(k * L, L)]
                qm = plsc.load_gather(qv, [cv])
                mk = qm > 0
                n = jnp.sum(mk.astype(jnp.int32))

                @pl.when(n > 0)
                def _():
                    dr = plsc.load_gather(disv, [rv])
                    dc = plsc.load_gather(disv, [cv])
                    plsc.store_compressed(lrow.at[pl.ds(cnt, L)], rv, mask=mk)
                    plsc.store_compressed(lsc.at[pl.ds(cnt, L)], dr * dc,
                                          mask=mk)
                    plsc.store_compressed(ltg.at[pl.ds(cnt, L)], qm, mask=mk)
                cnt = cnt + n
            return cnt
        cnt = lax.fori_loop(0, WIN, cstep, cnt)

        @pl.when(cnt > FLUSH_AT)
        def _():
            _drain_c(lrow, lsc, ltg, h1_hbm, rows, tgt128, agg2_acc, cnt,
                     sem_g, sem_s)
        return jnp.where(cnt > FLUSH_AT, 0, cnt)
    cnt = lax.fori_loop(0, NWIN, win, jnp.int32(0))

    base = wid * (NP // NW)

    def sstep(t, cnt):
        off = base + t * L
        qm = qv[pl.ds(off, L)]
        mk = qm > 0
        n = jnp.sum(mk.astype(jnp.int32))

        @pl.when(n > 0)
        def _():
            dv = disv[pl.ds(off, L)]
            vv = lax.iota(jnp.int32, L) + off
            plsc.store_compressed(lrow.at[pl.ds(cnt, L)], vv, mask=mk)
            plsc.store_compressed(lsc.at[pl.ds(cnt, L)], dv * dv, mask=mk)
            plsc.store_compressed(ltg.at[pl.ds(cnt, L)], qm, mask=mk)
        return cnt + n
    cnt = lax.fori_loop(0, (NP // NW) // L, sstep, cnt)

    _drain_c(lrow, lsc, ltg, h1_hbm, rows, tgt128, agg2_acc, cnt, sem_g, sem_s)
    plsc.subcore_barrier()

    @pl.when(sid == 0)
    def _():
        pltpu.sync_copy(agg2_acc, agg2_out.at[cid])


def _passC(row2d, col2d, dis, qmap, h1):
    f = pl.kernel(
        _passC_body,
        out_type=jax.ShapeDtypeStruct((NC, 8, D), jnp.float32),
        mesh=_mesh(),
        name="sc_passC",
        compiler_params=pltpu.CompilerParams(needs_layout_passes=False),
        scratch_types=[
            pltpu.VMEM_SHARED((8, D), jnp.float32),
            pltpu.VMEM((NP,), jnp.float32),
            pltpu.VMEM((NP,), jnp.int32),
            pltpu.VMEM((WIN, 128), jnp.int32),
            pltpu.VMEM((WIN, 128), jnp.int32),
            pltpu.VMEM((LCAP_C,), jnp.int32),
            pltpu.VMEM((LCAP_C,), jnp.float32),
            pltpu.VMEM((LCAP_C,), jnp.int32),
            pltpu.VMEM((128, D), jnp.float32),
            pltpu.VMEM((128,), jnp.int32),
            pltpu.SemaphoreType.DMA,
            pltpu.SemaphoreType.DMA,
        ],
    )
    return f(row2d, col2d, dis, qmap, h1)


def _prep_body(degp_ref, markp_ref, qmap_ref, dis_ref, dismk_ref):
    node = (lax.broadcasted_iota(jnp.int32, (NROW, D), 0) * D
            + lax.broadcasted_iota(jnp.int32, (NROW, D), 1))
    deg = degp_ref[0] + degp_ref[1] + 1.0
    dis = lax.rsqrt(deg)
    dis_ref[...] = dis
    m = ((markp_ref[0] + markp_ref[1]) > 0) | (qmap_ref[...] > 0)
    dismk_ref[...] = jnp.where(m & (node < N_NODES), dis, -dis)


def _prep(deg_parts, mark_parts, qmap):
    return pl.pallas_call(
        _prep_body,
        out_shape=[jax.ShapeDtypeStruct((NROW, D), jnp.float32),
                   jax.ShapeDtypeStruct((NROW, D), jnp.float32)],
    )(deg_parts.reshape(NC, NROW, D), mark_parts.reshape(NC, NROW, D), qmap)


def _h1_body(agg_ref, w_ref, b_ref, out_ref):
    agg = agg_ref[0] + agg_ref[1]
    acc = lax.dot_general(agg, w_ref[...],
                          (((1,), (1,)), ((), ())),
                          preferred_element_type=jnp.float32)
    out_ref[...] = jnp.maximum(acc + b_ref[...], 0.0)


def _h1_matmul(agg_parts, W1, b1):
    blk = 1024
    grid = NP // blk
    return pl.pallas_call(
        _h1_body,
        grid=(grid,),
        in_specs=[
            pl.BlockSpec((NC, blk, D), lambda i: (0, i, 0)),
            pl.BlockSpec((D, D), lambda i: (0, 0)),
            pl.BlockSpec((1, D), lambda i: (0, 0)),
        ],
        out_specs=pl.BlockSpec((blk, D), lambda i: (i, 0)),
        out_shape=jax.ShapeDtypeStruct((NP, D), jnp.float32),
    )(agg_parts, W1, b1.reshape(1, D))


def _head_body(agg2_ref, w2_ref, b2_ref, wp1a_ref, wp1b_ref, bp1_ref,
               wp2_ref, bp2_ref, xp_ref, out_ref):
    agg2 = agg2_ref[0] + agg2_ref[1]
    h2 = lax.dot_general(agg2, w2_ref[...], (((1,), (1,)), ((), ())),
                         preferred_element_type=jnp.float32)
    h2 = jnp.maximum(h2 + b2_ref[...], 0.0)
    p = lax.dot_general(xp_ref[...], wp1b_ref[...], (((1,), (1,)), ((), ())),
                        preferred_element_type=jnp.float32)
    for j in range(5):
        p = p + lax.dot_general(h2[j:j + 1, :], wp1a_ref[:, j * D:(j + 1) * D],
                                (((1,), (1,)), ((), ())),
                                preferred_element_type=jnp.float32)
    p = jnp.maximum(p + bp1_ref[...], 0.0)
    out = lax.dot_general(p, wp2_ref[...], (((1,), (1,)), ((), ())),
                          preferred_element_type=jnp.float32)
    out_ref[...] = out + bp2_ref[...]


def _head(agg2_parts, W2, b2, Wp1, bp1, Wp2, bp2, x):
    wp1a = Wp1[:, :640]
    wp1b = jnp.zeros((256, D), jnp.float32).at[:, :64].set(Wp1[:, 640:704])
    xp = jnp.zeros((1, D), jnp.float32).at[0, :64].set(x)
    out = pl.pallas_call(
        _head_body,
        out_shape=jax.ShapeDtypeStruct((1, 256), jnp.float32),
    )(agg2_parts, W2, b2.reshape(1, -1),
      wp1a, wp1b, bp1.reshape(1, -1), Wp2, bp2.reshape(1, -1), xp)
    return out.reshape(-1)


def kernel(graph_x, graph_edge_index, position, available_moves, x,
           W1, b1, W2, b2, Wp1, bp1, Wp2, bp2):
    row = graph_edge_index[0]
    col = graph_edge_index[1]
    row2d = jnp.concatenate([row, jnp.zeros((EPAD,), jnp.int32)]).reshape(EROWS, 128)
    col2d = jnp.concatenate([col, jnp.full((EPAD,), DUMP, jnp.int32)]).reshape(EROWS, 128)
    q8 = jnp.concatenate([available_moves, position,
                          jnp.full((3,), -1, jnp.int32)])

    qmap2d = _qmap(q8)
    deg_parts, mark_parts = _passA(row2d, col2d, qmap2d.reshape(NP))
    dis2d, dismk2d = _prep(deg_parts, mark_parts, qmap2d)
    dis = dis2d.reshape(NP)
    dismk = dismk2d.reshape(NP)
    qmap = qmap2d.reshape(NP)

    agg_parts = _passB(row2d, col2d, dismk, graph_x)
    h1 = _h1_matmul(agg_parts, W1, b1)

    agg2_parts = _passC(row2d, col2d, dis, qmap, h1)

    return _head(agg2_parts, W2, b2, Wp1, bp1, Wp2, bp2, x)


# speedup vs baseline: 1.7001x; 1.7001x over previous
import functools

import jax
import jax.numpy as jnp
from jax import lax
from jax.experimental import pallas as pl
from jax.experimental.pallas import tpu as pltpu
from jax.experimental.pallas import tpu_sc as plsc

N_NODES = 10000
N_EDGES = 320000
D = 128
NP = 10240
NROW = NP // 128
DUMP = 10200

NC, NS, L = 2, 16, 16
NW = NC * NS
WROWS = 80
EROWS = NW * WROWS
EPAD = EROWS * 128 - N_EDGES
SLICE = NP // NS


def _mesh():
    return plsc.VectorSubcoreMesh(core_axis_name="c", subcore_axis_name="s",
                                  num_cores=NC, num_subcores=NS)


def _qmap_body(q_ref, out_ref):
    node = (lax.broadcasted_iota(jnp.int32, (NROW, D), 0) * D
            + lax.broadcasted_iota(jnp.int32, (NROW, D), 1))
    acc = jnp.zeros((NROW, D), jnp.int32)
    for j in range(5):
        acc = acc | ((node == q_ref[j]).astype(jnp.int32) << j)
    out_ref[...] = acc


def _qmap(q8):
    return pl.pallas_call(
        _qmap_body,
        in_specs=[pl.BlockSpec(memory_space=pltpu.SMEM)],
        out_shape=jax.ShapeDtypeStruct((NROW, D), jnp.int32),
    )(q8)


def _passA_body(row2d, col2d, qmap_hbm, deg_out, mark_out,
                deg_acc, mark_acc, qv, rowb, colb, c128, idxb, onesb, zb,
                sem_d, sem_m):
    cid = lax.axis_index("c")
    sid = lax.axis_index("s")
    wid = cid * NS + sid

    ones16 = jnp.ones((L,), jnp.float32)
    zero16 = jnp.zeros((L,), jnp.float32)
    for k in range(8):
        onesb[pl.ds(k * L, L)] = ones16

    def zfill(i, c):
        zb[pl.ds(i * L, L)] = zero16
        return c
    lax.fori_loop(0, SLICE // L, zfill, 0)
    pltpu.sync_copy(zb, deg_acc.at[pl.ds(sid * SLICE, SLICE)])
    pltpu.sync_copy(zb, mark_acc.at[pl.ds(sid * SLICE, SLICE)])
    pltpu.sync_copy(qmap_hbm, qv)
    plsc.subcore_barrier()

    rb = wid * WROWS
    pltpu.sync_copy(row2d.at[pl.ds(rb, WROWS)], rowb)
    pltpu.sync_copy(col2d.at[pl.ds(rb, WROWS)], colb)

    def body(r, carry):
        cnt = jnp.int32(0)
        for k in range(8):
            cv = colb[r, pl.ds(k * L, L)]
            rv = rowb[r, pl.ds(k * L, L)]
            c128[pl.ds(k * L, L)] = cv
            hit = plsc.load_gather(qv, [cv]) > 0
            idxb[pl.ds(k * L, L)] = jnp.where(hit, rv, DUMP)
            cnt = cnt + jnp.sum(hit.astype(jnp.int32))
        cp = pltpu.async_copy(onesb, deg_acc.at[c128], sem_d, add=True)

        @pl.when(cnt > 0)
        def _():
            pltpu.async_copy(onesb, mark_acc.at[idxb], sem_m, add=True).wait()
        cp.wait()
        return carry
    lax.fori_loop(0, WROWS, body, 0)
    plsc.subcore_barrier()

    sl = pl.ds(sid * SLICE, SLICE)
    osl = pl.ds(cid * NP + sid * SLICE, SLICE)
    pltpu.sync_copy(deg_acc.at[sl], deg_out.at[osl])
    pltpu.sync_copy(mark_acc.at[sl], mark_out.at[osl])


def _passA(row2d, col2d, qmap):
    f = pl.kernel(
        _passA_body,
        out_type=[jax.ShapeDtypeStruct((NC * NP,), jnp.float32),
                  jax.ShapeDtypeStruct((NC * NP,), jnp.float32)],
        mesh=_mesh(),
        name="sc_passA",
        compiler_params=pltpu.CompilerParams(needs_layout_passes=False),
        scratch_types=[
            pltpu.VMEM_SHARED((NP,), jnp.float32),
            pltpu.VMEM_SHARED((NP,), jnp.float32),
            pltpu.VMEM((NP,), jnp.int32),
            pltpu.VMEM((WROWS, 128), jnp.int32),
            pltpu.VMEM((WROWS, 128), jnp.int32),
            pltpu.VMEM((128,), jnp.int32),
            pltpu.VMEM((128,), jnp.int32),
            pltpu.VMEM((128,), jnp.float32),
            pltpu.VMEM((SLICE,), jnp.float32),
            pltpu.SemaphoreType.DMA,
            pltpu.SemaphoreType.DMA,
        ],
    )
    return f(row2d, col2d, qmap)


LCAP_B = 4096
FLUSH_B = LCAP_B - 16 * 128 - 320 - 64
WIN = 16
NWIN = WROWS // WIN


def _drain(lrow, lsc, ltg, src_hbm, rows, tgtb, acc, cnt, sem_g, sem_s,
           dump_tg, br):
    zero16 = jnp.zeros((L,), jnp.float32)
    dump16 = jnp.full((L,), dump_tg, jnp.int32)
    izero16 = jnp.zeros((L,), jnp.int32)
    for k in range(br // L):
        lrow[pl.ds(cnt + k * L, L)] = izero16
        lsc[pl.ds(cnt + k * L, L)] = zero16
        ltg[pl.ds(cnt + k * L, L)] = dump16

    nb = (cnt + br - 1) // br

    def gstep(b, c):
        cp = pltpu.async_copy(src_hbm.at[lrow.at[pl.ds(b * br, br)]],
                              rows, sem_g)
        for k in range(br // L):
            tgtb[pl.ds(k * L, L)] = ltg[pl.ds(b * br + k * L, L)]
        cp.wait()

        def scale_row(i, cc):
            sb = plsc.load_gather(lsc, [jnp.full((L,), b * br + i, jnp.int32)])
            for k in range(8):
                rows[i, pl.ds(k * L, L)] = rows[i, pl.ds(k * L, L)] * sb
            return cc
        lax.fori_loop(0, br, scale_row, 0)
        pltpu.async_copy(rows, acc.at[tgtb], sem_s, add=True).wait()
        return c
    lax.fori_loop(0, nb, gstep, 0)


def _zero_rows(rows, nr):
    zero16 = jnp.zeros((L,), jnp.float32)

    def zf(i, c):
        for k in range(8):
            rows[i, pl.ds(k * L, L)] = zero16
        return c
    lax.fori_loop(0, nr, zf, 0)


def _passB_body(row2d, col2d, dismk_hbm, gx_hbm, agg_out,
                agg_acc, dmv, rowb, colb, lrow, lsc, ltg, rows, tgtb,
                sem_g, sem_s):
    cid = lax.axis_index("c")
    sid = lax.axis_index("s")
    wid = cid * NS + sid

    _zero_rows(rows, 64)
    pltpu.sync_copy(dismk_hbm, dmv)

    dump16 = jnp.full((L,), DUMP, jnp.int32)

    def zscan(t, cnt):
        off = sid * SLICE + t * L
        mk = dmv[pl.ds(off, L)] > 0
        n = jnp.sum(mk.astype(jnp.int32))

        @pl.when(n > 0)
        def _():
            vv = lax.iota(jnp.int32, L) + off
            plsc.store_compressed(lrow.at[pl.ds(cnt, L)], vv, mask=mk)
        return cnt + n
    cnt0 = lax.fori_loop(0, SLICE // L, zscan, jnp.int32(0))
    for k in range(4):
        lrow[pl.ds(cnt0 + k * L, L)] = dump16

    def zstep(b, c):
        for k in range(4):
            tgtb[pl.ds(k * L, L)] = lrow[pl.ds(b * 64 + k * L, L)]
        pltpu.async_copy(rows, agg_acc.at[tgtb], sem_s).wait()
        return c
    lax.fori_loop(0, (cnt0 + 63) // 64, zstep, 0)
    plsc.subcore_barrier()

    rb = wid * WROWS

    def win(w, cnt):
        pltpu.sync_copy(row2d.at[pl.ds(rb + w * WIN, WIN)], rowb)
        pltpu.sync_copy(col2d.at[pl.ds(rb + w * WIN, WIN)], colb)

        def cstep(r, cnt):
            for k in range(8):
                cv = colb[r, pl.ds(k * L, L)]
                rv = rowb[r, pl.ds(k * L, L)]
                dc = plsc.load_gather(dmv, [cv])
                mk = dc > 0
                n = jnp.sum(mk.astype(jnp.int32))

                @pl.when(n > 0)
                def _():
                    dr = jnp.abs(plsc.load_gather(dmv, [rv]))
                    plsc.store_compressed(lrow.at[pl.ds(cnt, L)], rv, mask=mk)
                    plsc.store_compressed(lsc.at[pl.ds(cnt, L)], dr * dc,
                                          mask=mk)
                    plsc.store_compressed(ltg.at[pl.ds(cnt, L)], cv, mask=mk)
                cnt = cnt + n
            return cnt
        cnt = lax.fori_loop(0, WIN, cstep, cnt)

        @pl.when(cnt > FLUSH_B)
        def _():
            _drain(lrow, lsc, ltg, gx_hbm, rows, tgtb, agg_acc, cnt,
                   sem_g, sem_s, DUMP, 64)
        return jnp.where(cnt > FLUSH_B, 0, cnt)
    cnt = lax.fori_loop(0, NWIN, win, jnp.int32(0))

    base = wid * (NP // NW)

    def sstep(t, cnt):
        off = base + t * L
        dv = dmv[pl.ds(off, L)]
        mk = dv > 0
        n = jnp.sum(mk.astype(jnp.int32))

        @pl.when(n > 0)
        def _():
            vv = lax.iota(jnp.int32, L) + off
            plsc.store_compressed(lrow.at[pl.ds(cnt, L)], vv, mask=mk)
            plsc.store_compressed(lsc.at[pl.ds(cnt, L)], dv * dv, mask=mk)
            plsc.store_compressed(ltg.at[pl.ds(cnt, L)], vv, mask=mk)
        return cnt + n
    cnt = lax.fori_loop(0, (NP // NW) // L, sstep, cnt)

    _drain(lrow, lsc, ltg, gx_hbm, rows, tgtb, agg_acc, cnt, sem_g, sem_s,
           DUMP, 64)
    plsc.subcore_barrier()
    for t in range(SLICE // 64):
        sl = pl.ds(sid * SLICE + t * 64, 64)
        pltpu.sync_copy(agg_acc.at[sl, :], agg_out.at[cid, sl, :])


def _passB(row2d, col2d, dismk, graph_x):
    f = pl.kernel(
        _passB_body,
        out_type=jax.ShapeDtypeStruct((NC, NP, D), jnp.float32),
        mesh=_mesh(),
        name="sc_passB",
        compiler_params=pltpu.CompilerParams(needs_layout_passes=False),
        scratch_types=[
            pltpu.VMEM_SHARED((NP, D), jnp.float32),
            pltpu.VMEM((NP,), jnp.float32),
            pltpu.VMEM((WIN, 128), jnp.int32),
            pltpu.VMEM((WIN, 128), jnp.int32),
            pltpu.VMEM((LCAP_B,), jnp.int32),
            pltpu.VMEM((LCAP_B,), jnp.float32),
            pltpu.VMEM((LCAP_B,), jnp.int32),
            pltpu.VMEM((64, D), jnp.float32),
            pltpu.VMEM((64,), jnp.int32),
            pltpu.SemaphoreType.DMA,
            pltpu.SemaphoreType.DMA,
        ],
    )
    return f(row2d, col2d, dismk, graph_x)


LCAP_C = 10752


def _passC_body(row2d, col2d, dis_hbm, qmap_hbm, h1_hbm, agg2_out,
                disv, qv, rowb, colb, lrow, lsc, lqm, rows, a2, sem_g):
    cid = lax.axis_index("c")
    sid = lax.axis_index("s")
    wid = cid * NS + sid

    zero16 = jnp.zeros((L,), jnp.float32)
    for j in range(8):
        for k in range(8):
            a2[j, pl.ds(k * L, L)] = zero16
    pltpu.sync_copy(dis_hbm, disv)
    pltpu.sync_copy(qmap_hbm, qv)

    rb = wid * WROWS

    def win(w, cnt):
        pltpu.sync_copy(row2d.at[pl.ds(rb + w * WIN, WIN)], rowb)
        pltpu.sync_copy(col2d.at[pl.ds(rb + w * WIN, WIN)], colb)

        def cstep(r, cnt):
            for k in range(8):
                cv = colb[r, pl.ds(k * L, L)]
                rv = rowb[r, pl.ds(k * L, L)]
                qm = plsc.load_gather(qv, [cv])
                mk = qm > 0
                n = jnp.sum(mk.astype(jnp.int32))

                @pl.when(n > 0)
                def _():
                    dr = plsc.load_gather(disv, [rv])
                    dc = plsc.load_gather(disv, [cv])
                    plsc.store_compressed(lrow.at[pl.ds(cnt, L)], rv, mask=mk)
                    plsc.store_compressed(lsc.at[pl.ds(cnt, L)], dr * dc,
                                          mask=mk)
                    plsc.store_compressed(lqm.at[pl.ds(cnt, L)], qm, mask=mk)
                cnt = cnt + n
            return cnt
        return lax.fori_loop(0, WIN, cstep, cnt)
    cnt = lax.fori_loop(0, NWIN, win, jnp.int32(0))

    base = wid * (NP // NW)

    def sstep(t, cnt):
        off = base + t * L
        qm = qv[pl.ds(off, L)]
        mk = qm > 0
        n = jnp.sum(mk.astype(jnp.int32))

        @pl.when(n > 0)
        def _():
            dv = disv[pl.ds(off, L)]
            vv = lax.iota(jnp.int32, L) + off
            plsc.store_compressed(lrow.at[pl.ds(cnt, L)], vv, mask=mk)
            plsc.store_compressed(lsc.at[pl.ds(cnt, L)], dv * dv, mask=mk)
            plsc.store_compressed(lqm.at[pl.ds(cnt, L)], qm, mask=mk)
        return cnt + n
    cnt = lax.fori_loop(0, (NP // NW) // L, sstep, cnt)

    izero16 = jnp.zeros((L,), jnp.int32)
    lrow[pl.ds(cnt, L)] = izero16
    lsc[pl.ds(cnt, L)] = zero16
    lqm[pl.ds(cnt, L)] = izero16

    nb = (cnt + L - 1) // L

    def gstep(b, c):
        pltpu.async_copy(h1_hbm.at[lrow.at[pl.ds(b * L, L)]], rows,
                         sem_g).wait()

        def acc_row(i, cc):
            gi = b * L + i
            sb = plsc.load_gather(lsc, [jnp.full((L,), gi, jnp.int32)])
            qb = plsc.load_gather(lqm, [jnp.full((L,), gi, jnp.int32)])
            for j in range(5):
                mj = ((qb >> j) & 1) > 0
                nj = jnp.sum(mj.astype(jnp.int32))

                @pl.when(nj > 0)
                def _():
                    for k in range(8):
                        a2[j, pl.ds(k * L, L)] = (
                            a2[j, pl.ds(k * L, L)]
                            + rows[i, pl.ds(k * L, L)] * sb)
            return cc
        lax.fori_loop(0, L, acc_row, 0)
        return c
    lax.fori_loop(0, nb, gstep, 0)

    pltpu.sync_copy(a2, agg2_out.at[wid])


def _passC(row2d, col2d, dis, qmap, h1):
    f = pl.kernel(
        _passC_body,
        out_type=jax.ShapeDtypeStruct((NW, 8, D), jnp.float32),
        mesh=_mesh(),
        name="sc_passC",
        compiler_params=pltpu.CompilerParams(needs_layout_passes=False),
        scratch_types=[
            pltpu.VMEM((NP,), jnp.float32),
            pltpu.VMEM((NP,), jnp.int32),
            pltpu.VMEM((WIN, 128), jnp.int32),
            pltpu.VMEM((WIN, 128), jnp.int32),
            pltpu.VMEM((LCAP_C,), jnp.int32),
            pltpu.VMEM((LCAP_C,), jnp.float32),
            pltpu.VMEM((LCAP_C,), jnp.int32),
            pltpu.VMEM((L, D), jnp.float32),
            pltpu.VMEM((8, D), jnp.float32),
            pltpu.SemaphoreType.DMA,
        ],
    )
    return f(row2d, col2d, dis, qmap, h1)


def _prep_body(degp_ref, markp_ref, qmap_ref, dis_ref, dismk_ref):
    node = (lax.broadcasted_iota(jnp.int32, (NROW, D), 0) * D
            + lax.broadcasted_iota(jnp.int32, (NROW, D), 1))
    deg = degp_ref[0] + degp_ref[1] + 1.0
    dis = lax.rsqrt(deg)
    dis_ref[...] = dis
    m = ((markp_ref[0] + markp_ref[1]) > 0) | (qmap_ref[...] > 0)
    dismk_ref[...] = jnp.where(m & (node < N_NODES), dis, -dis)


def _prep(deg_parts, mark_parts, qmap):
    return pl.pallas_call(
        _prep_body,
        out_shape=[jax.ShapeDtypeStruct((NROW, D), jnp.float32),
                   jax.ShapeDtypeStruct((NROW, D), jnp.float32)],
    )(deg_parts.reshape(NC, NROW, D), mark_parts.reshape(NC, NROW, D), qmap)


def _h1_body(agg_ref, w_ref, b_ref, out_ref):
    agg = agg_ref[0] + agg_ref[1]
    acc = lax.dot_general(agg, w_ref[...],
                          (((1,), (1,)), ((), ())),
                          preferred_element_type=jnp.float32)
    out_ref[...] = jnp.maximum(acc + b_ref[...], 0.0)


def _h1_matmul(agg_parts, W1, b1):
    blk = 1024
    grid = NP // blk
    return pl.pallas_call(
        _h1_body,
        grid=(grid,),
        in_specs=[
            pl.BlockSpec((NC, blk, D), lambda i: (0, i, 0)),
            pl.BlockSpec((D, D), lambda i: (0, 0)),
            pl.BlockSpec((1, D), lambda i: (0, 0)),
        ],
        out_specs=pl.BlockSpec((blk, D), lambda i: (i, 0)),
        out_shape=jax.ShapeDtypeStruct((NP, D), jnp.float32),
    )(agg_parts, W1, b1.reshape(1, D))


def _head_body(agg2_ref, w2_ref, b2_ref, wp1a_ref, wp1b_ref, bp1_ref,
               wp2_ref, bp2_ref, xp_ref, out_ref):
    agg2 = agg2_ref[0]
    for i in range(1, NW):
        agg2 = agg2 + agg2_ref[i]
    h2 = lax.dot_general(agg2, w2_ref[...], (((1,), (1,)), ((), ())),
                         preferred_element_type=jnp.float32)
    h2 = jnp.maximum(h2 + b2_ref[...], 0.0)
    p = lax.dot_general(xp_ref[...], wp1b_ref[...], (((1,), (1,)), ((), ())),
                        preferred_element_type=jnp.float32)
    for j in range(5):
        p = p + lax.dot_general(h2[j:j + 1, :], wp1a_ref[:, j * D:(j + 1) * D],
                                (((1,), (1,)), ((), ())),
                                preferred_element_type=jnp.float32)
    p = jnp.maximum(p + bp1_ref[...], 0.0)
    out = lax.dot_general(p, wp2_ref[...], (((1,), (1,)), ((), ())),
                          preferred_element_type=jnp.float32)
    out_ref[...] = out + bp2_ref[...]


def _head(agg2_parts, W2, b2, Wp1, bp1, Wp2, bp2, x):
    wp1a = Wp1[:, :640]
    wp1b = jnp.zeros((256, D), jnp.float32).at[:, :64].set(Wp1[:, 640:704])
    xp = jnp.zeros((1, D), jnp.float32).at[0, :64].set(x)
    out = pl.pallas_call(
        _head_body,
        out_shape=jax.ShapeDtypeStruct((1, 256), jnp.float32),
    )(agg2_parts, W2, b2.reshape(1, -1),
      wp1a, wp1b, bp1.reshape(1, -1), Wp2, bp2.reshape(1, -1), xp)
    return out.reshape(-1)


def kernel(graph_x, graph_edge_index, position, available_moves, x,
           W1, b1, W2, b2, Wp1, bp1, Wp2, bp2):
    row = graph_edge_index[0]
    col = graph_edge_index[1]
    row2d = jnp.concatenate([row, jnp.zeros((EPAD,), jnp.int32)]).reshape(EROWS, 128)
    col2d = jnp.concatenate([col, jnp.full((EPAD,), DUMP, jnp.int32)]).reshape(EROWS, 128)
    q8 = jnp.concatenate([available_moves, position,
                          jnp.full((3,), -1, jnp.int32)])

    qmap2d = _qmap(q8)
    deg_parts, mark_parts = _passA(row2d, col2d, qmap2d.reshape(NP))
    dis2d, dismk2d = _prep(deg_parts, mark_parts, qmap2d)
    dis = dis2d.reshape(NP)
    dismk = dismk2d.reshape(NP)
    qmap = qmap2d.reshape(NP)

    agg_parts = _passB(row2d, col2d, dismk, graph_x)
    h1 = _h1_matmul(agg_parts, W1, b1)

    agg2_parts = _passC(row2d, col2d, dis, qmap, h1)

    return _head(agg2_parts, W2, b2, Wp1, bp1, Wp2, bp2, x)
